# baseline (speedup 1.0000x reference)
import jax
import jax.numpy as jnp
from jax.experimental import pallas as pl

_BLOCK_E = 5000


def _rgcn_block_kernel(x_ref, cs_ref, w_ref, o_ref):
    wsum = jnp.sum(w_ref[...], axis=1)
    csb = cs_ref[...].reshape(x_ref.shape[0], cs_ref.shape[2])
    a = jnp.dot(1.0 / csb, wsum, preferred_element_type=jnp.float32)
    o_ref[...] = jnp.sum(x_ref[...], axis=1, keepdims=True) * a


def kernel(x, edge_index, W, cs):
    del edge_index
    E, J = x.shape
    R, I, O = W.shape
    be = _BLOCK_E if E % _BLOCK_E == 0 else E
    grid = (E // be,)
    cs3 = cs.reshape(E // 8, 8, R)
    return pl.pallas_call(
        _rgcn_block_kernel,
        grid=grid,
        in_specs=[
            pl.BlockSpec((be, J), lambda i: (i, 0)),
            pl.BlockSpec((be // 8, 8, R), lambda i: (i, 0, 0)),
            pl.BlockSpec((R, I, O), lambda i: (0, 0, 0)),
        ],
        out_specs=pl.BlockSpec((be, O), lambda i: (i, 0)),
        out_shape=jax.ShapeDtypeStruct((E, O), jnp.float32),
    )(x, cs3, W)


# speedup vs baseline: 1.1537x; 1.1537x over previous
import jax
import jax.numpy as jnp
from jax.experimental import pallas as pl

_BLOCK_E = 5000


def _rgcn_block_kernel(x_ref, cst_ref, w_ref, o_ref):
    wsum = jnp.sum(w_ref[...], axis=1)
    recip_t = 1.0 / cst_ref[0].astype(jnp.float32)
    a = jax.lax.dot_general(
        recip_t, wsum,
        dimension_numbers=(((0,), (0,)), ((), ())),
        preferred_element_type=jnp.float32,
    )
    o_ref[...] = jnp.sum(x_ref[...], axis=1, keepdims=True) * a


def kernel(x, edge_index, W, cs):
    del edge_index
    E, J = x.shape
    R, I, O = W.shape
    be = _BLOCK_E if E % _BLOCK_E == 0 else E
    grid = (E // be,)
    cst = cs.reshape(E // be, be, R).transpose(0, 2, 1).astype(jnp.bfloat16)
    return pl.pallas_call(
        _rgcn_block_kernel,
        grid=grid,
        in_specs=[
            pl.BlockSpec((be, J), lambda i: (i, 0)),
            pl.BlockSpec((1, R, be), lambda i: (i, 0, 0)),
            pl.BlockSpec((R, I, O), lambda i: (0, 0, 0)),
        ],
        out_specs=pl.BlockSpec((be, O), lambda i: (i, 0)),
        out_shape=jax.ShapeDtypeStruct((E, O), jnp.float32),
    )(x, cst, W)
